# baseline (speedup 1.0000x reference)
import jax
import jax.numpy as jnp
from jax.experimental import pallas as pl
from jax.experimental.pallas import tpu as pltpu

_THRESHOLD = 0.5
_H = 128
_W = 256
_N = 5
_C = 256
_HT = 8


def _fused_kernel(x_ref, psm_ref, gk_ref, out_ref, cnt_ref):
    i = pl.program_id(0)
    tile = psm_ref[:, :, pl.ds(i * _HT, _HT + 4), :]
    maps = jnp.max(jax.nn.sigmoid(tile), axis=1)
    acc = None
    for dy in range(5):
        for dx in range(5):
            term = maps[:, dy:dy + _HT, dx:dx + _W] * gk_ref[dy, dx]
            acc = term if acc is None else acc + term
    m = jnp.where(acc > _THRESHOLD, 1.0, 0.0).astype(jnp.float32)
    cnt_ref[...] = jnp.sum(m).reshape(1, 1, 1)
    cav_idx = jax.lax.broadcasted_iota(jnp.int32, m.shape, 0)
    m = jnp.where(cav_idx == 0, 1.0, m)

    x0 = x_ref[0]
    inv_sqrt_c = 0.0625
    scores = [jnp.sum(x0 * x0, axis=0) * inv_sqrt_c]
    for cav in range(1, _N):
        d = jnp.sum(x0 * x_ref[cav], axis=0) * inv_sqrt_c
        scores.append(d * m[cav])
    smax = scores[0]
    for cav in range(1, _N):
        smax = jnp.maximum(smax, scores[cav])
    exps = [jnp.exp(s - smax) for s in scores]
    denom = exps[0]
    for cav in range(1, _N):
        denom = denom + exps[cav]
    inv_denom = 1.0 / denom
    out = x0 * (exps[0] * inv_denom)
    for cav in range(1, _N):
        out = out + x_ref[cav] * (exps[cav] * inv_denom * m[cav])
    out_ref[...] = out


def kernel(x, psm_single, record_len, pairwise_t_matrix, trajectory, gauss_kernel):
    del record_len, pairwise_t_matrix, trajectory
    psm_p = jnp.pad(psm_single, ((0, 0), (0, 0), (2, 2), (2, 2)),
                    constant_values=-1e30)

    grid = (_H // _HT,)
    x_fuse, counts = pl.pallas_call(
        _fused_kernel,
        grid=grid,
        in_specs=[
            pl.BlockSpec((_N, _C, _HT, _W), lambda i: (0, 0, i, 0)),
            pl.BlockSpec((_N, 2, _H + 4, _W + 4), lambda i: (0, 0, 0, 0)),
            pl.BlockSpec(memory_space=pltpu.SMEM),
        ],
        out_specs=(
            pl.BlockSpec((_C, _HT, _W), lambda i: (0, i, 0)),
            pl.BlockSpec((1, 1, 1), lambda i: (i, 0, 0)),
        ),
        out_shape=(
            jax.ShapeDtypeStruct((_C, _H, _W), jnp.float32),
            jax.ShapeDtypeStruct((grid[0], 1, 1), jnp.float32),
        ),
        compiler_params=pltpu.CompilerParams(
            dimension_semantics=("parallel",),
        ),
    )(x, psm_p, gauss_kernel)

    rate = jnp.sum(counts) / (_N * _H * _W)
    return x_fuse[None], rate


# speedup vs baseline: 1.0193x; 1.0193x over previous
import jax
import jax.numpy as jnp
from jax.experimental import pallas as pl
from jax.experimental.pallas import tpu as pltpu

_THRESHOLD = 0.5
_H = 128
_W = 256
_N = 5
_C = 256
_HT = 8


def _mask_kernel(psm_ref, gk_ref, mask_ref, rate_ref):
    maps = jnp.max(jax.nn.sigmoid(psm_ref[...]), axis=1)
    acc = None
    for dy in range(5):
        for dx in range(5):
            term = maps[:, dy:dy + _H, dx:dx + _W] * gk_ref[dy, dx]
            acc = term if acc is None else acc + term
    mask = jnp.where(acc > _THRESHOLD, 1.0, 0.0).astype(jnp.float32)
    rate_ref[0, 0] = jnp.sum(mask) / (_N * _H * _W)
    cav_idx = jax.lax.broadcasted_iota(jnp.int32, mask.shape, 0)
    mask_ref[...] = jnp.where(cav_idx == 0, 1.0, mask)


def _fuse_kernel(x_ref, mask_ref, out_ref):
    m = mask_ref[...]
    x0 = x_ref[0]
    inv_sqrt_c = 0.0625
    scores = [jnp.sum(x0 * x0, axis=0) * inv_sqrt_c]
    for cav in range(1, _N):
        d = jnp.sum(x0 * x_ref[cav], axis=0) * inv_sqrt_c
        scores.append(d * m[cav])
    smax = scores[0]
    for cav in range(1, _N):
        smax = jnp.maximum(smax, scores[cav])
    exps = [jnp.exp(s - smax) for s in scores]
    denom = exps[0]
    for cav in range(1, _N):
        denom = denom + exps[cav]
    inv_denom = 1.0 / denom
    out = x0 * (exps[0] * inv_denom)
    for cav in range(1, _N):
        out = out + x_ref[cav] * (exps[cav] * inv_denom * m[cav])
    out_ref[...] = out


def kernel(x, psm_single, record_len, pairwise_t_matrix, trajectory, gauss_kernel):
    del record_len, pairwise_t_matrix, trajectory
    psm_p = jnp.pad(psm_single, ((0, 0), (0, 0), (2, 2), (2, 2)),
                    constant_values=-1e30)

    mask, rate = pl.pallas_call(
        _mask_kernel,
        out_shape=(
            jax.ShapeDtypeStruct((_N, _H, _W), jnp.float32),
            jax.ShapeDtypeStruct((1, 1), jnp.float32),
        ),
        in_specs=[
            pl.BlockSpec((_N, 2, _H + 4, _W + 4), lambda: (0, 0, 0, 0)),
            pl.BlockSpec(memory_space=pltpu.SMEM),
        ],
        out_specs=(
            pl.BlockSpec((_N, _H, _W), lambda: (0, 0, 0)),
            pl.BlockSpec(memory_space=pltpu.SMEM),
        ),
    )(psm_p, gauss_kernel)

    grid = (_H // _HT,)
    x_fuse = pl.pallas_call(
        _fuse_kernel,
        grid=grid,
        in_specs=[
            pl.BlockSpec((_N, _C, _HT, _W), lambda i: (0, 0, i, 0)),
            pl.BlockSpec((_N, _HT, _W), lambda i: (0, i, 0)),
        ],
        out_specs=pl.BlockSpec((_C, _HT, _W), lambda i: (0, i, 0)),
        out_shape=jax.ShapeDtypeStruct((_C, _H, _W), jnp.float32),
        compiler_params=pltpu.CompilerParams(
            dimension_semantics=("parallel",),
        ),
    )(x, mask)

    return x_fuse[None], rate[0, 0]
